# baseline (speedup 1.0000x reference)
import functools

import jax
import jax.numpy as jnp
from jax import lax
from jax.experimental import pallas as pl
from jax.experimental.pallas import tpu as pltpu
from jax.experimental.pallas import tpu_sc as plsc

NENTITY = 1_000_000
NREL = 1000
D = 64
B = 16384
NNEG = 200
L = 16
NC, NS = 2, 16
NW = NC * NS
RPW = B // NW
CB = 4
NSTEPS = RPW // CB
GROUPS = (NNEG + L - 1) // L
W = D // 2
CHUNK = CB * NNEG
GCH = 100
NGD = CHUNK // GCH


def _body(ent_hbm, rel_hbm, hidx_hbm, ridx_hbm, tidx_hbm, nidx_hbm,
          pos_hbm, neg_hbm,
          hidx_v, ridx_v, tidx_v, posbuf, hrex_e, hrex_o,
          prow_a, prow_b, rrow_a, rrow_b, trow_a, trow_b,
          nidx_a, nidx_b, rows_a, rows_b, nout_a, nout_b,
          psem_a, psem_b, sem_a, sem_b, osem_a, osem_b, isem_a, isem_b):
    isem_ = (isem_a, isem_b)
    prow_ = (prow_a, prow_b)
    rrow_ = (rrow_a, rrow_b)
    trow_ = (trow_a, trow_b)
    nidx_ = (nidx_a, nidx_b)
    rows_ = (rows_a, rows_b)
    nout_ = (nout_a, nout_b)
    psem_ = (psem_a, psem_b)
    nsem_ = (sem_a, sem_b)
    osem_ = (osem_a, osem_b)
    wid = lax.axis_index("s") * NC + lax.axis_index("c")
    base = wid * RPW
    pltpu.sync_copy(hidx_hbm.at[pl.ds(base, RPW)], hidx_v.at[pl.ds(0, RPW)])
    pltpu.sync_copy(ridx_hbm.at[pl.ds(base, RPW)], ridx_v.at[pl.ds(0, RPW)])
    pltpu.sync_copy(tidx_hbm.at[pl.ds(base, RPW)], tidx_v.at[pl.ds(0, RPW)])
    iota = lax.iota(jnp.int32, L)
    zero16 = jnp.zeros((L,), jnp.int32)
    hidx_v[pl.ds(RPW, L)] = zero16
    ridx_v[pl.ds(RPW, L)] = zero16
    tidx_v[pl.ds(RPW, L)] = zero16

    def stage_nidx(s, nb):
        r0s = jnp.minimum(s, NSTEPS - 1) * CB
        pltpu.sync_copy(nidx_hbm.at[pl.ds((base + r0s) * (NNEG // GCH), NGD)],
                        nb)

    def stage_nidx_async(s, nb, sem):
        r0s = jnp.minimum(s, NSTEPS - 1) * CB
        pltpu.async_copy(nidx_hbm.at[pl.ds((base + r0s) * (NNEG // GCH), NGD)],
                         nb, sem)

    def wait_nidx(nb, sem):
        pltpu.make_async_copy(nidx_hbm.at[pl.ds(0, NGD)], nb, sem).wait()

    def issue_gathers(nb, rows, sem):
        return [pltpu.async_copy(ent_hbm.at[nb.at[j]],
                                 rows.at[pl.ds(j * GCH, GCH)], sem)
                for j in range(NGD)]

    def wait_gathers(nb, rows, sem):
        for j in range(NGD):
            pltpu.make_async_copy(ent_hbm.at[nb.at[j]],
                                  rows.at[pl.ds(j * GCH, GCH)], sem).wait()

    def issue_pos(h, nx):
        r0 = h * CB
        hv = hidx_v[pl.ds(r0, L)]
        rv = ridx_v[pl.ds(r0, L)]
        tv = tidx_v[pl.ds(r0, L)]
        pltpu.async_copy(ent_hbm.at[hv], prow_[nx], psem_[nx])
        pltpu.async_copy(rel_hbm.at[rv], rrow_[nx], psem_[nx])
        pltpu.async_copy(ent_hbm.at[tv], trow_[nx], psem_[nx])

    def unpk(words):
        return plsc.unpack(plsc.bitcast(words, jnp.bfloat16),
                           format=plsc.PackFormat.INTERLEAVED)

    def pos_compute(h, cur):
        r0 = h * CB
        prow, rrow, trow = prow_[cur], rrow_[cur], trow_[cur]
        for ref in (prow, rrow, trow):
            tbl = ent_hbm if ref is not rrow else rel_hbm
            pltpu.make_async_copy(tbl.at[iota], ref, psem_[cur]).wait()
        psc = jnp.zeros((L,), jnp.float32)
        for i in range(CB):
            acc = jnp.zeros((L,), jnp.float32)
            for half in range(W // L):
                he, ho = unpk(prow[i, pl.ds(half * L, L)])
                re_, ro = unpk(rrow[i, pl.ds(half * L, L)])
                te, to = unpk(trow[i, pl.ds(half * L, L)])
                hre = he * re_
                hro = ho * ro
                acc = acc + hre * te + hro * to
                hrex_e[i, pl.ds(half * L, L)] = hre
                hrex_e[i, pl.ds(W + half * L, L)] = hre
                hrex_o[i, pl.ds(half * L, L)] = hro
                hrex_o[i, pl.ds(W + half * L, L)] = hro
            psc = jnp.where(iota == i, jnp.sum(acc), psc)
        plsc.store_scatter(posbuf, [jnp.minimum(r0 + iota, RPW - 1)], psc,
                           mask=iota < CB)

    def neg_compute(h, rows_v, nout_v):
        zf = jnp.zeros((L,), jnp.float32)
        for i in range(CB):
            for gb in range(3):
                nbase = i * NNEG + gb * 4 * L
                ids = [nbase + gg * L + iota for gg in range(4)]

                def dbody(m, carry, i=i, ids=ids, rows_v=rows_v):
                    a0, a1, a2, a3, col = carry
                    hbe = hrex_e[i, pl.ds(m, L)]
                    hbo = hrex_o[i, pl.ds(m, L)]
                    a = [a0, a1, a2, a3]
                    for gg in range(4):
                        ve, vo = unpk(plsc.load_gather(rows_v,
                                                       [ids[gg], col]))
                        a[gg] = a[gg] + hbe * ve + hbo * vo
                    return (a[0], a[1], a[2], a[3],
                            (col + 1) & (W - 1))

                a0, a1, a2, a3, _ = plsc.parallel_loop(
                    0, W, carry=(zf, zf, zf, zf, iota))(dbody)
                for gg, agg in enumerate((a0, a1, a2, a3)):
                    plsc.store_scatter(nout_v, [ids[gg]], agg)
            pos0 = i * NNEG + 12 * L
            ids_t = jnp.minimum(pos0 + iota, CHUNK - 1)
            mask_t = (pos0 + iota) < (i + 1) * NNEG

            def tbody(m, carry, i=i, ids_t=ids_t, rows_v=rows_v):
                acc, col = carry
                hbe = hrex_e[i, pl.ds(m, L)]
                hbo = hrex_o[i, pl.ds(m, L)]
                ve, vo = unpk(plsc.load_gather(rows_v, [ids_t, col]))
                return (acc + hbe * ve + hbo * vo, (col + 1) & (W - 1))

            acc_t, _ = plsc.parallel_loop(0, W, carry=(zf, iota))(tbody)
            plsc.store_scatter(nout_v, [ids_t], acc_t, mask=mask_t)

    def wait_nout(cur):
        pltpu.make_async_copy(nout_[cur],
                              neg_hbm.at[pl.ds(0, CHUNK)], osem_[cur]).wait()

    def substep(p, h, cur, last_issue_guard):
        nx = 1 - cur
        def _issue():
            wait_nidx(nidx_[nx], isem_[nx])
            issue_gathers(nidx_[nx], rows_[nx], nsem_[nx])
            issue_pos(h + 1, nx)
        if last_issue_guard is None:
            _issue()
        else:
            pl.when(last_issue_guard)(_issue)
        pos_compute(h, cur)
        wait_gathers(nidx_[cur], rows_[cur], nsem_[cur])
        stage_nidx_async(h + 2, nidx_[cur], isem_[cur])
        pl.when(p > 0)(lambda: wait_nout(cur))
        neg_compute(h, rows_[cur], nout_[cur])
        pltpu.async_copy(nout_[cur],
                         neg_hbm.at[pl.ds((base + h * CB) * NNEG, CHUNK)],
                         osem_[cur])

    stage_nidx(0, nidx_a)
    issue_gathers(nidx_a, rows_a, sem_a)
    issue_pos(0, 0)
    stage_nidx_async(1, nidx_b, isem_b)

    def pair(p, carry):
        substep(p, 2 * p, 0, None)
        substep(p, 2 * p + 1, 1, p < NSTEPS // 2 - 1)
        return carry

    lax.fori_loop(0, NSTEPS // 2, pair, 0)
    wait_nout(0)
    wait_nout(1)
    wait_nidx(nidx_a, isem_a)
    wait_nidx(nidx_b, isem_b)
    pltpu.sync_copy(posbuf, pos_hbm.at[pl.ds(base, RPW)])


@functools.partial(
    pl.kernel,
    out_type=(jax.ShapeDtypeStruct((B,), jnp.float32),
              jax.ShapeDtypeStruct((B * NNEG,), jnp.float32)),
    mesh=plsc.VectorSubcoreMesh(core_axis_name="c", subcore_axis_name="s",
                                num_cores=NC, num_subcores=NS),
    compiler_params=pltpu.CompilerParams(needs_layout_passes=False,
                                         use_tc_tiling_on_sc=False),
    scratch_types=[
        pltpu.VMEM((RPW + L,), jnp.int32),
        pltpu.VMEM((RPW + L,), jnp.int32),
        pltpu.VMEM((RPW + L,), jnp.int32),
        pltpu.VMEM((RPW,), jnp.float32),
        pltpu.VMEM((CB, 2 * W), jnp.float32),
        pltpu.VMEM((CB, 2 * W), jnp.float32),
        pltpu.VMEM((L, W), jnp.int32),
        pltpu.VMEM((L, W), jnp.int32),
        pltpu.VMEM((L, W), jnp.int32),
        pltpu.VMEM((L, W), jnp.int32),
        pltpu.VMEM((L, W), jnp.int32),
        pltpu.VMEM((L, W), jnp.int32),
        pltpu.VMEM((NGD, GCH), jnp.int32),
        pltpu.VMEM((NGD, GCH), jnp.int32),
        pltpu.VMEM((CHUNK, W), jnp.int32),
        pltpu.VMEM((CHUNK, W), jnp.int32),
        pltpu.VMEM((CHUNK,), jnp.float32),
        pltpu.VMEM((CHUNK,), jnp.float32),
        pltpu.SemaphoreType.DMA,
        pltpu.SemaphoreType.DMA,
        pltpu.SemaphoreType.DMA,
        pltpu.SemaphoreType.DMA,
        pltpu.SemaphoreType.DMA,
        pltpu.SemaphoreType.DMA,
        pltpu.SemaphoreType.DMA,
        pltpu.SemaphoreType.DMA,
    ],
)
def _distmult_sc(ent_hbm, rel_hbm, hidx_hbm, ridx_hbm, tidx_hbm, nidx_hbm,
                 pos_hbm, neg_hbm, *scratch):
    _body(ent_hbm, rel_hbm, hidx_hbm, ridx_hbm, tidx_hbm, nidx_hbm,
          pos_hbm, neg_hbm, *scratch)


def _pack_bf16(table):
    bf = table.astype(jnp.bfloat16)
    return jax.lax.bitcast_convert_type(
        bf.reshape(table.shape[0], W, 2), jnp.int32)


def kernel(positive, negative, entity_embedding, relation_embedding):
    hidx = positive[:, 0].astype(jnp.int32)
    ridx = positive[:, 1].astype(jnp.int32)
    tidx = positive[:, 2].astype(jnp.int32)
    nidx = negative.astype(jnp.int32).reshape(B * NNEG // GCH, GCH)
    pos, negf = _distmult_sc(_pack_bf16(entity_embedding),
                             _pack_bf16(relation_embedding),
                             hidx, ridx, tidx, nidx)
    return pos, negf.reshape(B, NNEG)


# speedup vs baseline: 1.8049x; 1.8049x over previous
import functools

import jax
import jax.numpy as jnp
from jax import lax
from jax.experimental import pallas as pl
from jax.experimental.pallas import tpu as pltpu
from jax.experimental.pallas import tpu_sc as plsc

NENTITY = 1_000_000
NREL = 1000
D = 64
B = 16384
NNEG = 200
L = 16
NC, NS = 2, 16
NW = NC * NS
RPW = B // NW
CB = 4
NSTEPS = RPW // CB
GROUPS = (NNEG + L - 1) // L
CHUNK = CB * NNEG
GCH = 100
NGD = CHUNK // GCH


def _body(ent_hbm, rel_hbm, hidx_hbm, ridx_hbm, tidx_hbm, nidx_hbm,
          pos_hbm, neg_hbm,
          hidx_v, ridx_v, tidx_v, posbuf, hrext,
          prow_a, rrow_a, trow_a,
          nidx_a, nidx_b, rows_a, rows_b, nout_a, nout_b,
          psem_a, sem_a, sem_b, osem_a, osem_b, isem_a, isem_b):
    isem_ = (isem_a, isem_b)
    nidx_ = (nidx_a, nidx_b)
    rows_ = (rows_a, rows_b)
    nout_ = (nout_a, nout_b)
    nsem_ = (sem_a, sem_b)
    osem_ = (osem_a, osem_b)
    wid = lax.axis_index("s") * NC + lax.axis_index("c")
    base = wid * RPW
    pltpu.sync_copy(hidx_hbm.at[pl.ds(base, RPW)], hidx_v.at[pl.ds(0, RPW)])
    pltpu.sync_copy(ridx_hbm.at[pl.ds(base, RPW)], ridx_v.at[pl.ds(0, RPW)])
    pltpu.sync_copy(tidx_hbm.at[pl.ds(base, RPW)], tidx_v.at[pl.ds(0, RPW)])
    iota = lax.iota(jnp.int32, L)
    zero16 = jnp.zeros((L,), jnp.int32)
    hidx_v[pl.ds(RPW, L)] = zero16
    ridx_v[pl.ds(RPW, L)] = zero16
    tidx_v[pl.ds(RPW, L)] = zero16

    def stage_nidx(s, nb):
        r0s = jnp.minimum(s, NSTEPS - 1) * CB
        pltpu.sync_copy(nidx_hbm.at[pl.ds((base + r0s) * (NNEG // GCH), NGD)],
                        nb)

    def stage_nidx_async(s, nb, sem):
        r0s = jnp.minimum(s, NSTEPS - 1) * CB
        pltpu.async_copy(nidx_hbm.at[pl.ds((base + r0s) * (NNEG // GCH), NGD)],
                         nb, sem)

    def wait_nidx(nb, sem):
        pltpu.make_async_copy(nidx_hbm.at[pl.ds(0, NGD)], nb, sem).wait()

    def issue_gathers(nb, rows, sem):
        return [pltpu.async_copy(ent_hbm.at[nb.at[j]],
                                 rows.at[pl.ds(j * GCH, GCH)], sem)
                for j in range(NGD)]

    def wait_gathers(nb, rows, sem):
        for j in range(NGD):
            pltpu.make_async_copy(ent_hbm.at[nb.at[j]],
                                  rows.at[pl.ds(j * GCH, GCH)], sem).wait()

    def issue_pos(r0):
        hv = hidx_v[pl.ds(r0, L)]
        rv = ridx_v[pl.ds(r0, L)]
        tv = tidx_v[pl.ds(r0, L)]
        pltpu.async_copy(ent_hbm.at[hv], prow_a, psem_a)
        pltpu.async_copy(rel_hbm.at[rv], rrow_a, psem_a)
        pltpu.async_copy(ent_hbm.at[tv], trow_a, psem_a)

    def wait_pos():
        for ref in (prow_a, rrow_a, trow_a):
            tbl = ent_hbm if ref is not rrow_a else rel_hbm
            pltpu.make_async_copy(tbl.at[iota], ref, psem_a).wait()

    def pos_compute(h):
        r0 = h * CB
        rbase = (h % 4) * CB
        psc = jnp.zeros((L,), jnp.float32)
        for i in range(CB):
            acc = jnp.zeros((L,), jnp.float32)
            for k in range(D // L):
                hrk = (prow_a[rbase + i, pl.ds(k * L, L)]
                       * rrow_a[rbase + i, pl.ds(k * L, L)])
                acc = acc + hrk * trow_a[rbase + i, pl.ds(k * L, L)]
                hrext[i, pl.ds(k * L, L)] = hrk
                hrext[i, pl.ds(D + k * L, L)] = hrk
            psc = jnp.where(iota == i, jnp.sum(acc), psc)
        plsc.store_scatter(posbuf, [jnp.minimum(r0 + iota, RPW - 1)], psc,
                           mask=iota < CB)

    def neg_compute(h, rows_v, nout_v):
        zf = jnp.zeros((L,), jnp.float32)
        for i in range(CB):
            for gb in range(3):
                nbase = i * NNEG + gb * 4 * L
                ids = [nbase + gg * L + iota for gg in range(4)]

                def dbody(d, carry, i=i, ids=ids, rows_v=rows_v):
                    a0, a1, a2, a3, col = carry
                    hb = hrext[i, pl.ds(d, L)]
                    a = [a0, a1, a2, a3]
                    for gg in range(4):
                        v = plsc.load_gather(rows_v, [ids[gg], col])
                        a[gg] = a[gg] + hb * v
                    return (a[0], a[1], a[2], a[3],
                            (col + 1) & (D - 1))

                a0, a1, a2, a3, _ = plsc.parallel_loop(
                    0, D, carry=(zf, zf, zf, zf, iota))(dbody)
                for gg, agg in enumerate((a0, a1, a2, a3)):
                    plsc.store_scatter(nout_v, [ids[gg]], agg)
            pos0 = i * NNEG + 12 * L
            ids_t = jnp.minimum(pos0 + iota, CHUNK - 1)
            mask_t = (pos0 + iota) < (i + 1) * NNEG

            def tbody(d, carry, i=i, ids_t=ids_t, rows_v=rows_v):
                acc, col = carry
                hb = hrext[i, pl.ds(d, L)]
                v = plsc.load_gather(rows_v, [ids_t, col])
                return (acc + hb * v, (col + 1) & (D - 1))

            acc_t, _ = plsc.parallel_loop(0, D, carry=(zf, iota))(tbody)
            plsc.store_scatter(nout_v, [ids_t], acc_t, mask=mask_t)

    def wait_nout(cur):
        pltpu.make_async_copy(nout_[cur],
                              neg_hbm.at[pl.ds(0, CHUNK)], osem_[cur]).wait()

    def substep(p, h, cur, last_issue_guard):
        nx = 1 - cur
        def _issue():
            wait_nidx(nidx_[nx], isem_[nx])
            issue_gathers(nidx_[nx], rows_[nx], nsem_[nx])
        if last_issue_guard is None:
            _issue()
        else:
            pl.when(last_issue_guard)(_issue)
        pl.when(h % 4 == 0)(wait_pos)
        pos_compute(h)
        pl.when(h % 4 == 3)(lambda: issue_pos((h + 1) * CB))
        wait_gathers(nidx_[cur], rows_[cur], nsem_[cur])
        stage_nidx_async(h + 2, nidx_[cur], isem_[cur])
        pl.when(p > 0)(lambda: wait_nout(cur))
        neg_compute(h, rows_[cur], nout_[cur])
        pltpu.async_copy(nout_[cur],
                         neg_hbm.at[pl.ds((base + h * CB) * NNEG, CHUNK)],
                         osem_[cur])

    stage_nidx(0, nidx_a)
    issue_gathers(nidx_a, rows_a, sem_a)
    issue_pos(0)
    stage_nidx_async(1, nidx_b, isem_b)

    def pair(p, carry):
        substep(p, 2 * p, 0, None)
        substep(p, 2 * p + 1, 1, p < NSTEPS // 2 - 1)
        return carry

    lax.fori_loop(0, NSTEPS // 2, pair, 0)
    wait_nout(0)
    wait_nout(1)
    wait_nidx(nidx_a, isem_a)
    wait_nidx(nidx_b, isem_b)
    wait_pos()
    pltpu.sync_copy(posbuf, pos_hbm.at[pl.ds(base, RPW)])


@functools.partial(
    pl.kernel,
    out_type=(jax.ShapeDtypeStruct((B,), jnp.float32),
              jax.ShapeDtypeStruct((B * NNEG,), jnp.float32)),
    mesh=plsc.VectorSubcoreMesh(core_axis_name="c", subcore_axis_name="s",
                                num_cores=NC, num_subcores=NS),
    compiler_params=pltpu.CompilerParams(needs_layout_passes=False,
                                         use_tc_tiling_on_sc=False),
    scratch_types=[
        pltpu.VMEM((RPW + L,), jnp.int32),
        pltpu.VMEM((RPW + L,), jnp.int32),
        pltpu.VMEM((RPW + L,), jnp.int32),
        pltpu.VMEM((RPW,), jnp.float32),
        pltpu.VMEM((CB, 2 * D), jnp.float32),
        pltpu.VMEM((L, D), jnp.float32),
        pltpu.VMEM((L, D), jnp.float32),
        pltpu.VMEM((L, D), jnp.float32),
        pltpu.VMEM((NGD, GCH), jnp.int32),
        pltpu.VMEM((NGD, GCH), jnp.int32),
        pltpu.VMEM((CHUNK, D), jnp.float32),
        pltpu.VMEM((CHUNK, D), jnp.float32),
        pltpu.VMEM((CHUNK,), jnp.float32),
        pltpu.VMEM((CHUNK,), jnp.float32),
        pltpu.SemaphoreType.DMA,
        pltpu.SemaphoreType.DMA,
        pltpu.SemaphoreType.DMA,
        pltpu.SemaphoreType.DMA,
        pltpu.SemaphoreType.DMA,
        pltpu.SemaphoreType.DMA,
        pltpu.SemaphoreType.DMA,
    ],
)
def _distmult_sc(ent_hbm, rel_hbm, hidx_hbm, ridx_hbm, tidx_hbm, nidx_hbm,
                 pos_hbm, neg_hbm, *scratch):
    _body(ent_hbm, rel_hbm, hidx_hbm, ridx_hbm, tidx_hbm, nidx_hbm,
          pos_hbm, neg_hbm, *scratch)


def kernel(positive, negative, entity_embedding, relation_embedding):
    hidx = positive[:, 0].astype(jnp.int32)
    ridx = positive[:, 1].astype(jnp.int32)
    tidx = positive[:, 2].astype(jnp.int32)
    nidx = negative.astype(jnp.int32).reshape(B * NNEG // GCH, GCH)
    pos, negf = _distmult_sc(entity_embedding, relation_embedding,
                             hidx, ridx, tidx, nidx)
    return pos, negf.reshape(B, NNEG)


# speedup vs baseline: 1.8250x; 1.0111x over previous
import functools

import jax
import jax.numpy as jnp
from jax import lax
from jax.experimental import pallas as pl
from jax.experimental.pallas import tpu as pltpu
from jax.experimental.pallas import tpu_sc as plsc

NENTITY = 1_000_000
NREL = 1000
D = 64
B = 16384
NNEG = 200
L = 16
NC, NS = 2, 16
NW = NC * NS
RPW = B // NW
CB = 4
NSTEPS = RPW // CB
GROUPS = (NNEG + L - 1) // L
CHUNK = CB * NNEG
GCH = 100
NGD = CHUNK // GCH


def _body(ent_hbm, rel_hbm, pidx_hbm, nidx_hbm,
          pos_hbm, neg_hbm,
          pidx_v, posbuf, hrext,
          prow_a, rrow_a, trow_a,
          nidx_a, nidx_b, rows_a, rows_b, nout_a, nout_b,
          psem_a, sem_a, sem_b, osem_a, osem_b, isem_a, isem_b):
    isem_ = (isem_a, isem_b)
    nidx_ = (nidx_a, nidx_b)
    rows_ = (rows_a, rows_b)
    nout_ = (nout_a, nout_b)
    nsem_ = (sem_a, sem_b)
    osem_ = (osem_a, osem_b)
    wid = lax.axis_index("s") * NC + lax.axis_index("c")
    base = wid * RPW
    pltpu.sync_copy(pidx_hbm.at[pl.ds(base, RPW)], pidx_v)
    iota = lax.iota(jnp.int32, L)

    def stage_nidx(s, nb):
        r0s = jnp.minimum(s, NSTEPS - 1) * CB
        pltpu.sync_copy(nidx_hbm.at[pl.ds((base + r0s) * (NNEG // GCH), NGD)],
                        nb)

    def stage_nidx_async(s, nb, sem):
        r0s = jnp.minimum(s, NSTEPS - 1) * CB
        pltpu.async_copy(nidx_hbm.at[pl.ds((base + r0s) * (NNEG // GCH), NGD)],
                         nb, sem)

    def wait_nidx(nb, sem):
        pltpu.make_async_copy(nidx_hbm.at[pl.ds(0, NGD)], nb, sem).wait()

    def issue_gathers(nb, rows, sem):
        return [pltpu.async_copy(ent_hbm.at[nb.at[j]],
                                 rows.at[pl.ds(j * GCH, GCH)], sem)
                for j in range(NGD)]

    def wait_gathers(nb, rows, sem):
        for j in range(NGD):
            pltpu.make_async_copy(ent_hbm.at[nb.at[j]],
                                  rows.at[pl.ds(j * GCH, GCH)], sem).wait()

    def issue_pos(r0):
        sel = jnp.minimum(r0 + iota, RPW - 1)
        hv = plsc.load_gather(pidx_v, [sel, jnp.full((L,), 0, jnp.int32)])
        rv = plsc.load_gather(pidx_v, [sel, jnp.full((L,), 1, jnp.int32)])
        tv = plsc.load_gather(pidx_v, [sel, jnp.full((L,), 2, jnp.int32)])
        pltpu.async_copy(ent_hbm.at[hv], prow_a, psem_a)
        pltpu.async_copy(rel_hbm.at[rv], rrow_a, psem_a)
        pltpu.async_copy(ent_hbm.at[tv], trow_a, psem_a)

    def wait_pos():
        for ref in (prow_a, rrow_a, trow_a):
            tbl = ent_hbm if ref is not rrow_a else rel_hbm
            pltpu.make_async_copy(tbl.at[iota], ref, psem_a).wait()

    def pos_compute(h):
        r0 = h * CB
        rbase = (h % 4) * CB
        psc = jnp.zeros((L,), jnp.float32)
        for i in range(CB):
            acc = jnp.zeros((L,), jnp.float32)
            for k in range(D // L):
                hrk = (prow_a[rbase + i, pl.ds(k * L, L)]
                       * rrow_a[rbase + i, pl.ds(k * L, L)])
                acc = acc + hrk * trow_a[rbase + i, pl.ds(k * L, L)]
                hrext[i, pl.ds(k * L, L)] = hrk
                hrext[i, pl.ds(D + k * L, L)] = hrk
            psc = jnp.where(iota == i, jnp.sum(acc), psc)
        plsc.store_scatter(posbuf, [jnp.minimum(r0 + iota, RPW - 1)], psc,
                           mask=iota < CB)

    def neg_compute(h, rows_v, nout_v):
        zf = jnp.zeros((L,), jnp.float32)
        for i in range(CB):
            for gb in range(3):
                nbase = i * NNEG + gb * 4 * L
                ids = [nbase + gg * L + iota for gg in range(4)]

                def dbody(d, carry, i=i, ids=ids, rows_v=rows_v):
                    a0, a1, a2, a3, col = carry
                    hb = hrext[i, pl.ds(d, L)]
                    a = [a0, a1, a2, a3]
                    for gg in range(4):
                        v = plsc.load_gather(rows_v, [ids[gg], col])
                        a[gg] = a[gg] + hb * v
                    return (a[0], a[1], a[2], a[3],
                            (col + 1) & (D - 1))

                a0, a1, a2, a3, _ = plsc.parallel_loop(
                    0, D, carry=(zf, zf, zf, zf, iota))(dbody)
                for gg, agg in enumerate((a0, a1, a2, a3)):
                    plsc.store_scatter(nout_v, [ids[gg]], agg)
            pos0 = i * NNEG + 12 * L
            ids_t = jnp.minimum(pos0 + iota, CHUNK - 1)
            mask_t = (pos0 + iota) < (i + 1) * NNEG

            def tbody(d, carry, i=i, ids_t=ids_t, rows_v=rows_v):
                acc, col = carry
                hb = hrext[i, pl.ds(d, L)]
                v = plsc.load_gather(rows_v, [ids_t, col])
                return (acc + hb * v, (col + 1) & (D - 1))

            acc_t, _ = plsc.parallel_loop(0, D, carry=(zf, iota))(tbody)
            plsc.store_scatter(nout_v, [ids_t], acc_t, mask=mask_t)

    def wait_nout(cur):
        pltpu.make_async_copy(nout_[cur],
                              neg_hbm.at[pl.ds(0, CHUNK)], osem_[cur]).wait()

    def substep(p, h, cur, last_issue_guard):
        nx = 1 - cur
        def _issue():
            wait_nidx(nidx_[nx], isem_[nx])
            issue_gathers(nidx_[nx], rows_[nx], nsem_[nx])
        if last_issue_guard is None:
            _issue()
        else:
            pl.when(last_issue_guard)(_issue)
        pl.when(h % 4 == 0)(wait_pos)
        pos_compute(h)
        pl.when(h % 4 == 3)(lambda: issue_pos((h + 1) * CB))
        wait_gathers(nidx_[cur], rows_[cur], nsem_[cur])
        stage_nidx_async(h + 2, nidx_[cur], isem_[cur])
        pl.when(p > 0)(lambda: wait_nout(cur))
        neg_compute(h, rows_[cur], nout_[cur])
        pltpu.async_copy(nout_[cur],
                         neg_hbm.at[pl.ds((base + h * CB) * NNEG, CHUNK)],
                         osem_[cur])

    stage_nidx(0, nidx_a)
    issue_gathers(nidx_a, rows_a, sem_a)
    issue_pos(0)
    stage_nidx_async(1, nidx_b, isem_b)

    def pair(p, carry):
        substep(p, 2 * p, 0, None)
        substep(p, 2 * p + 1, 1, p < NSTEPS // 2 - 1)
        return carry

    lax.fori_loop(0, NSTEPS // 2, pair, 0)
    wait_nout(0)
    wait_nout(1)
    wait_nidx(nidx_a, isem_a)
    wait_nidx(nidx_b, isem_b)
    wait_pos()
    pltpu.sync_copy(posbuf, pos_hbm.at[pl.ds(base, RPW)])


@functools.partial(
    pl.kernel,
    out_type=(jax.ShapeDtypeStruct((B,), jnp.float32),
              jax.ShapeDtypeStruct((B * NNEG,), jnp.float32)),
    mesh=plsc.VectorSubcoreMesh(core_axis_name="c", subcore_axis_name="s",
                                num_cores=NC, num_subcores=NS),
    compiler_params=pltpu.CompilerParams(needs_layout_passes=False,
                                         use_tc_tiling_on_sc=False),
    scratch_types=[
        pltpu.VMEM((RPW, 3), jnp.int32),
        pltpu.VMEM((RPW,), jnp.float32),
        pltpu.VMEM((CB, 2 * D), jnp.float32),
        pltpu.VMEM((L, D), jnp.float32),
        pltpu.VMEM((L, D), jnp.float32),
        pltpu.VMEM((L, D), jnp.float32),
        pltpu.VMEM((NGD, GCH), jnp.int32),
        pltpu.VMEM((NGD, GCH), jnp.int32),
        pltpu.VMEM((CHUNK, D), jnp.float32),
        pltpu.VMEM((CHUNK, D), jnp.float32),
        pltpu.VMEM((CHUNK,), jnp.float32),
        pltpu.VMEM((CHUNK,), jnp.float32),
        pltpu.SemaphoreType.DMA,
        pltpu.SemaphoreType.DMA,
        pltpu.SemaphoreType.DMA,
        pltpu.SemaphoreType.DMA,
        pltpu.SemaphoreType.DMA,
        pltpu.SemaphoreType.DMA,
        pltpu.SemaphoreType.DMA,
    ],
)
def _distmult_sc(ent_hbm, rel_hbm, pidx_hbm, nidx_hbm,
                 pos_hbm, neg_hbm, *scratch):
    _body(ent_hbm, rel_hbm, pidx_hbm, nidx_hbm,
          pos_hbm, neg_hbm, *scratch)


def kernel(positive, negative, entity_embedding, relation_embedding):
    nidx = negative.astype(jnp.int32).reshape(B * NNEG // GCH, GCH)
    pos, negf = _distmult_sc(entity_embedding, relation_embedding,
                             positive.astype(jnp.int32), nidx)
    return pos, negf.reshape(B, NNEG)
